# Initial kernel scaffold; baseline (speedup 1.0000x reference)
import jax
import jax.numpy as jnp
from jax.experimental import pallas as pl


def kernel(x, edge_index, batch, edge_weight, emb_W, emb_b, conv_lin_W, ern_W1, ern_b1, ern_W2, ern_b2, diss_W, diss_b, mlp_W1, mlp_b1, mlp_W2, mlp_b2, ro_W1, ro_b1, ro_W2, ro_b2):
    raise NotImplementedError("write your pallas kernel here")


# speedup vs baseline: 1.6120x; 1.6120x over previous
import functools

import jax
import jax.numpy as jnp
from jax import lax
from jax.experimental import pallas as pl
from jax.experimental.pallas import tpu as pltpu
from jax.experimental.pallas import tpu_sc as plsc

N = 10000
E = 320000
HID = 256
OUT_DIM = 64
NUM_BLOCKS = 2
NUM_ITERS = 2
EPS = 0.1

NC = 2
NS = 16
NW = NC * NS

E_PAD = 327680
EPW = E_PAD // NW
CH1 = 64
NCH1 = EPW // CH1
EPT = E_PAD // NS
CH2 = 128
NCH2 = EPT // CH2
HH = HID // 2

BN = 2000
BM = 4096

_mesh = plsc.VectorSubcoreMesh(core_axis_name="c", subcore_axis_name="s")


def _dott(a, b):
    return lax.dot_general(a, b, (((1,), (1,)), ((), ())),
                           preferred_element_type=jnp.float32)


def _lin_kernel(x_ref, w_ref, b_ref, o_ref):
    o_ref[...] = _dott(x_ref[...], w_ref[...]) + b_ref[...]


def _tc_linear(x, w, b, bm):
    m, k = x.shape
    o = w.shape[0]
    return pl.pallas_call(
        _lin_kernel,
        grid=(m // bm,),
        in_specs=[pl.BlockSpec((bm, k), lambda i: (i, 0)),
                  pl.BlockSpec((o, k), lambda i: (0, 0)),
                  pl.BlockSpec((1, o), lambda i: (0, 0))],
        out_specs=pl.BlockSpec((bm, o), lambda i: (i, 0)),
        out_shape=jax.ShapeDtypeStruct((m, o), jnp.float32),
    )(x, w, b.reshape(1, o))


def _pq_kernel(h_ref, wa_ref, wb_ref, b1_ref, p_ref, q_ref):
    h = h_ref[...]
    p_ref[...] = _dott(h, wa_ref[...]) + b1_ref[...]
    q_ref[...] = _dott(h, wb_ref[...])


def _tc_pq(h, wa, wb, b1):
    return pl.pallas_call(
        _pq_kernel,
        grid=(N // BN,),
        in_specs=[pl.BlockSpec((BN, HID), lambda i: (i, 0)),
                  pl.BlockSpec((HID, HID), lambda i: (0, 0)),
                  pl.BlockSpec((HID, HID), lambda i: (0, 0)),
                  pl.BlockSpec((1, HID), lambda i: (0, 0))],
        out_specs=[pl.BlockSpec((BN, HID), lambda i: (i, 0)),
                   pl.BlockSpec((BN, HID), lambda i: (i, 0))],
        out_shape=[jax.ShapeDtypeStruct((N, HID), jnp.float32),
                   jax.ShapeDtypeStruct((N, HID), jnp.float32)],
    )(h, wa, wb, b1.reshape(1, HID))


def _feat_kernel(h_ref, cw_ref, dw_ref, db_ref, f2_ref, diss_ref):
    h = h_ref[...]
    f = _dott(h, cw_ref[...])
    f2_ref[0] = f[:, :HH]
    f2_ref[1] = f[:, HH:]
    diss_ref[...] = jnp.maximum(_dott(h, dw_ref[...]) + db_ref[...], 0.0)


def _tc_feat(h, cw, dw, db):
    return pl.pallas_call(
        _feat_kernel,
        grid=(N // BN,),
        in_specs=[pl.BlockSpec((BN, HID), lambda i: (i, 0)),
                  pl.BlockSpec((HID, HID), lambda i: (0, 0)),
                  pl.BlockSpec((HID, HID), lambda i: (0, 0)),
                  pl.BlockSpec((1, HID), lambda i: (0, 0))],
        out_specs=[pl.BlockSpec((2, BN, HH), lambda i: (0, i, 0)),
                   pl.BlockSpec((BN, HID), lambda i: (i, 0))],
        out_shape=[jax.ShapeDtypeStruct((2, N, HH), jnp.float32),
                   jax.ShapeDtypeStruct((N, HID), jnp.float32)],
    )(h, cw, dw, db.reshape(1, HID))


def _upd_kernel(h_ref, v_ref, f2_ref, s2_ref, degp_ref, diss_ref, ho_ref, vo_ref):
    in_feat = jnp.concatenate([f2_ref[0], f2_ref[1]], axis=1)
    s = jnp.concatenate([s2_ref[0], s2_ref[1]], axis=1)
    deg = degp_ref[0, :, :1] + degp_ref[1, :, :1]
    conv = deg * in_feat - s
    v = v_ref[...]
    v = v - EPS * (conv + diss_ref[...] * v)
    ho_ref[...] = h_ref[...] + EPS * v
    vo_ref[...] = v


def _tc_update(h, v, f2, s2, degp, diss):
    return pl.pallas_call(
        _upd_kernel,
        grid=(N // BN,),
        in_specs=[pl.BlockSpec((BN, HID), lambda i: (i, 0)),
                  pl.BlockSpec((BN, HID), lambda i: (i, 0)),
                  pl.BlockSpec((2, BN, HH), lambda i: (0, i, 0)),
                  pl.BlockSpec((2, BN, HH), lambda i: (0, i, 0)),
                  pl.BlockSpec((2, BN, 16), lambda i: (0, i, 0)),
                  pl.BlockSpec((BN, HID), lambda i: (i, 0))],
        out_specs=[pl.BlockSpec((BN, HID), lambda i: (i, 0)),
                   pl.BlockSpec((BN, HID), lambda i: (i, 0))],
        out_shape=[jax.ShapeDtypeStruct((N, HID), jnp.float32),
                   jax.ShapeDtypeStruct((N, HID), jnp.float32)],
    )(h, v, f2, s2, degp, diss)


def _mlp_kernel(h_ref, w1_ref, b1_ref, w2_ref, b2_ref, o_ref):
    t = jnp.tanh(_dott(h_ref[...], w1_ref[...]) + b1_ref[...])
    o_ref[...] = _dott(t, w2_ref[...]) + b2_ref[...]


def _tc_mlp(h, w1, b1, w2, b2):
    return pl.pallas_call(
        _mlp_kernel,
        grid=(N // BN,),
        in_specs=[pl.BlockSpec((BN, HID), lambda i: (i, 0)),
                  pl.BlockSpec((HID, HID), lambda i: (0, 0)),
                  pl.BlockSpec((1, HID), lambda i: (0, 0)),
                  pl.BlockSpec((HID, HID), lambda i: (0, 0)),
                  pl.BlockSpec((1, HID), lambda i: (0, 0))],
        out_specs=pl.BlockSpec((BN, HID), lambda i: (i, 0)),
        out_shape=jax.ShapeDtypeStruct((N, HID), jnp.float32),
    )(h, w1, b1.reshape(1, HID), w2, b2.reshape(1, HID))


def _lrelu(x):
    return jnp.where(x >= 0.0, x, 0.01 * x)


def _ro_kernel(h_ref, w1_ref, b1_ref, w2_ref, b2_ref, o_ref):
    t = _lrelu(_dott(h_ref[...], w1_ref[...]) + b1_ref[...])
    o_ref[...] = _lrelu(_dott(t, w2_ref[...]) + b2_ref[...])


def _tc_readout(h, w1, b1, w2, b2):
    hh, od = w1.shape[0], w2.shape[0]
    return pl.pallas_call(
        _ro_kernel,
        grid=(N // BN,),
        in_specs=[pl.BlockSpec((BN, HID), lambda i: (i, 0)),
                  pl.BlockSpec((hh, HID), lambda i: (0, 0)),
                  pl.BlockSpec((1, hh), lambda i: (0, 0)),
                  pl.BlockSpec((od, hh), lambda i: (0, 0)),
                  pl.BlockSpec((1, od), lambda i: (0, 0))],
        out_specs=pl.BlockSpec((BN, od), lambda i: (i, 0)),
        out_shape=jax.ShapeDtypeStruct((N, od), jnp.float32),
    )(h, w1, b1.reshape(1, hh), w2, b2.reshape(1, od))


_GDN = lax.GatherDimensionNumbers(offset_dims=(), collapsed_slice_dims=(0,),
                                  start_index_map=(0,))


def _vperm(v, idx):
    return lax.gather(v, idx.reshape(16, 1), _GDN, (1,),
                      mode=lax.GatherScatterMode.PROMISE_IN_BOUNDS)


def _lane_bcast(v, k):
    return _vperm(v, jnp.full((16,), k, jnp.int32))


def _allsum(v):
    lane = lax.iota(jnp.int32, 16)
    for sh in (8, 4, 2, 1):
        v = v + _vperm(v, lane ^ sh)
    return v


@functools.partial(
    pl.kernel, mesh=_mesh,
    compiler_params=pltpu.CompilerParams(needs_layout_passes=False,
                                         use_tc_tiling_on_sc=False),
    out_type=[jax.ShapeDtypeStruct((E_PAD,), jnp.float32),
              jax.ShapeDtypeStruct((NC, N, 16), jnp.float32)],
    scratch_types=[
        pltpu.VMEM((CH1,), jnp.int32),
        pltpu.VMEM((CH1,), jnp.int32),
        pltpu.VMEM((CH1, HID), jnp.float32),
        pltpu.VMEM((CH1, HID), jnp.float32),
        pltpu.VMEM((CH1, HID), jnp.float32),
        pltpu.VMEM((CH1,), jnp.float32),
        pltpu.VMEM((CH1, 16), jnp.float32),
        pltpu.VMEM((HID,), jnp.float32),
        pltpu.VMEM((16,), jnp.float32),
        pltpu.VMEM_SHARED((N, 16), jnp.float32),
        pltpu.SemaphoreType.DMA,
        pltpu.SemaphoreType.DMA,
    ])
def _sc_res(p_hbm, q_hbm, r_hbm, row_hbm, col_hbm, w2_hbm, b2_hbm, zdeg_hbm,
            er_out, degp_out,
            idxr, idxc, bufP, bufQ, bufR, bufE, bufD, w2v, b2v, degacc,
            sem1, sem2):
    c = lax.axis_index("c")
    s = lax.axis_index("s")
    wid = s * NC + c
    base = wid * EPW

    pltpu.sync_copy(w2_hbm, w2v)
    pltpu.sync_copy(b2_hbm, b2v)

    @pl.when(s == 0)
    def _():
        pltpu.sync_copy(zdeg_hbm.at[c], degacc)

    def _zb(i, carry):
        bufD[i, :] = jnp.zeros((16,), jnp.float32)
        return carry
    lax.fori_loop(0, CH1, _zb, 0)

    w2r = [w2v[pl.ds(t * 16, 16)] for t in range(16)]
    b2r = b2v[...]
    lane = lax.iota(jnp.int32, 16)
    m15 = lane == 15
    zer = jnp.zeros((16,), jnp.int32)

    plsc.subcore_barrier()

    def _chunk(j, carry):
        off = base + j * CH1
        pltpu.sync_copy(row_hbm.at[pl.ds(off, CH1)], idxr)
        pltpu.sync_copy(col_hbm.at[pl.ds(off, CH1)], idxc)
        cp = pltpu.async_copy(p_hbm.at[idxr], bufP, sem1)
        cq = pltpu.async_copy(q_hbm.at[idxc], bufQ, sem2)
        pltpu.sync_copy(r_hbm.at[pl.ds(off, CH1), :], bufR)
        cp.wait()
        cq.wait()

        def _edge(k, carry2):
            acc = jnp.zeros((16,), jnp.float32)
            for t in range(16):
                sl = pl.ds(t * 16, 16)
                z = bufP[k, sl] + bufQ[k, sl] + bufR[k, sl]
                z = jnp.maximum(z, 0.0)
                acc = acc + z * w2r[t]
            erv = jnp.abs(_allsum(acc) + b2r)
            validf = jnp.where(off + k < E, 1.0, 0.0).astype(jnp.float32)
            erv = erv * validf
            idxk = jnp.full((16,), k, jnp.int32)
            plsc.store_scatter(bufE, [idxk], erv, mask=m15)
            plsc.store_scatter(bufD, [idxk, zer], erv, mask=m15)
            return carry2
        lax.fori_loop(0, CH1, _edge, 0)

        pltpu.sync_copy(bufE, er_out.at[pl.ds(off, CH1)])
        pltpu.sync_copy(bufD, degacc.at[idxr], add=True)
        return carry
    lax.fori_loop(0, NCH1, _chunk, 0)

    plsc.subcore_barrier()

    @pl.when(s == 0)
    def _():
        pltpu.sync_copy(degacc, degp_out.at[c])


@functools.partial(
    pl.kernel, mesh=_mesh,
    compiler_params=pltpu.CompilerParams(needs_layout_passes=False,
                                         use_tc_tiling_on_sc=False),
    out_type=jax.ShapeDtypeStruct((NC, N, HH), jnp.float32),
    scratch_types=[
        pltpu.VMEM((CH2,), jnp.int32),
        pltpu.VMEM((CH2,), jnp.int32),
        pltpu.VMEM((CH2,), jnp.int32),
        pltpu.VMEM((CH2, HH), jnp.float32),
        pltpu.VMEM((CH2,), jnp.float32),
        pltpu.VMEM_SHARED((N, HH), jnp.float32),
        pltpu.SemaphoreType.DMA,
    ])
def _sc_scat(f2_hbm, er_hbm, row_hbm, col_hbm, zs_hbm,
             s2_out,
             idxr, idxro, idxc, bufG, bufE, sacc, sem):
    c = lax.axis_index("c")
    s = lax.axis_index("s")
    base = s * EPT
    coff = c * N

    @pl.when(s == 0)
    def _():
        pltpu.sync_copy(zs_hbm.at[c], sacc)

    plsc.subcore_barrier()

    def _chunk(j, carry):
        off = base + j * CH2
        pltpu.sync_copy(row_hbm.at[pl.ds(off, CH2)], idxr)
        pltpu.sync_copy(col_hbm.at[pl.ds(off, CH2)], idxc)

        def _oi(t, carry2):
            sl = pl.ds(t * 16, 16)
            idxro[sl] = idxr[sl] + coff
            return carry2
        lax.fori_loop(0, CH2 // 16, _oi, 0)

        pltpu.async_copy(f2_hbm.at[idxro], bufG, sem).wait()
        pltpu.sync_copy(er_hbm.at[pl.ds(off, CH2)], bufE)

        def _e16(t, carry2):
            ev = bufE[pl.ds(t * 16, 16)]
            for k in range(16):
                eb = _lane_bcast(ev, k)
                r = t * 16 + k
                for q in range(HH // 16):
                    sl = pl.ds(q * 16, 16)
                    bufG[r, sl] = bufG[r, sl] * eb
            return carry2
        lax.fori_loop(0, CH2 // 16, _e16, 0)

        pltpu.sync_copy(bufG, sacc.at[idxc], add=True)
        return carry
    lax.fori_loop(0, NCH2, _chunk, 0)

    plsc.subcore_barrier()

    @pl.when(s == 0)
    def _():
        pltpu.sync_copy(sacc, s2_out.at[c])


def kernel(x, edge_index, batch, edge_weight, emb_W, emb_b, conv_lin_W,
           ern_W1, ern_b1, ern_W2, ern_b2, diss_W, diss_b,
           mlp_W1, mlp_b1, mlp_W2, mlp_b2, ro_W1, ro_b1, ro_W2, ro_b2):
    pad = E_PAD - E
    rowp = jnp.pad(edge_index[0], (0, pad))
    colp = jnp.pad(edge_index[1], (0, pad))
    ew8 = jnp.pad(edge_weight, ((0, pad), (0, 4)))
    zdeg = jnp.zeros((NC, N, 16), jnp.float32)
    zs = jnp.zeros((NC, N, HH), jnp.float32)

    h = _tc_linear(x, emb_W, emb_b, BN)
    for i in range(NUM_BLOCKS):
        w1 = ern_W1[i]
        wa = w1[:, :HID]
        wb = w1[:, HID:2 * HID]
        wc8 = jnp.pad(w1[:, 2 * HID:], ((0, 0), (0, 4)))
        p, q = _tc_pq(h, wa, wb, ern_b1[i])
        r = _tc_linear(ew8, wc8, jnp.zeros((HID,), jnp.float32), BM)
        b2vec = jnp.broadcast_to(ern_b2[i, 0], (16,))
        er, degp = _sc_res(p, q, r, rowp, colp, ern_W2[i, 0], b2vec, zdeg)

        v = jnp.zeros((N, HID), jnp.float32)
        for _ in range(NUM_ITERS):
            f2, diss = _tc_feat(h, conv_lin_W[i], diss_W[i], diss_b[i])
            s2 = _sc_scat(f2.reshape(2 * N, HH), er, rowp, colp, zs)
            h, v = _tc_update(h, v, f2, s2, degp, diss)
        h = _tc_mlp(h, mlp_W1[i], mlp_b1[i], mlp_W2[i], mlp_b2[i])

    return _tc_readout(h, ro_W1, ro_b1, ro_W2, ro_b2)


# speedup vs baseline: 2.3954x; 1.4860x over previous
import functools

import jax
import jax.numpy as jnp
import numpy as np
from jax import lax
from jax.experimental import pallas as pl
from jax.experimental.pallas import tpu as pltpu
from jax.experimental.pallas import tpu_sc as plsc

N = 10000
E = 320000
HID = 256
OUT_DIM = 64
NUM_BLOCKS = 2
NUM_ITERS = 2
EPS = 0.1

NC = 2
NS = 16
NW = NC * NS

E_PAD = 327680
EPW = E_PAD // NW
CH1 = 128
NCH1 = EPW // CH1
EPT = E_PAD // NS
CH2 = 128
NCH2 = EPT // CH2
HH = HID // 2

BN = 2000
BM = 4096

_mesh = plsc.VectorSubcoreMesh(core_axis_name="c", subcore_axis_name="s")
_scp = pltpu.CompilerParams(needs_layout_passes=False,
                            use_tc_tiling_on_sc=False)

_k = np.arange(HID)
_m = _k % 32
_PERM = (_k - _m + 16 * (_m % 2) + _m // 2).astype(np.int32)
_PERM_W = (_k - _m + 2 * (_m % 16) + _m // 16).astype(np.int32)


def _dott(a, b):
    return lax.dot_general(a, b, (((1,), (1,)), ((), ())),
                           preferred_element_type=jnp.float32)


def _lin_kernel(x_ref, w_ref, b_ref, o_ref):
    y = _dott(x_ref[...], w_ref[...]) + b_ref[...]
    o_ref[...] = y.astype(o_ref.dtype)


def _tc_linear(x, w, b, bm, out_dtype=jnp.float32):
    m, k = x.shape
    o = w.shape[0]
    return pl.pallas_call(
        _lin_kernel,
        grid=(m // bm,),
        in_specs=[pl.BlockSpec((bm, k), lambda i: (i, 0)),
                  pl.BlockSpec((o, k), lambda i: (0, 0)),
                  pl.BlockSpec((1, o), lambda i: (0, 0))],
        out_specs=pl.BlockSpec((bm, o), lambda i: (i, 0)),
        out_shape=jax.ShapeDtypeStruct((m, o), out_dtype),
    )(x, w, b.reshape(1, o))


def _pq_kernel(h_ref, wa_ref, wb_ref, b1_ref, p_ref, q_ref):
    h = h_ref[...]
    p_ref[...] = (_dott(h, wa_ref[...]) + b1_ref[...]).astype(jnp.bfloat16)
    q_ref[...] = _dott(h, wb_ref[...]).astype(jnp.bfloat16)


def _tc_pq(h, wa, wb, b1):
    return pl.pallas_call(
        _pq_kernel,
        grid=(N // BN,),
        in_specs=[pl.BlockSpec((BN, HID), lambda i: (i, 0)),
                  pl.BlockSpec((HID, HID), lambda i: (0, 0)),
                  pl.BlockSpec((HID, HID), lambda i: (0, 0)),
                  pl.BlockSpec((1, HID), lambda i: (0, 0))],
        out_specs=[pl.BlockSpec((BN, HID), lambda i: (i, 0)),
                   pl.BlockSpec((BN, HID), lambda i: (i, 0))],
        out_shape=[jax.ShapeDtypeStruct((N, HID), jnp.bfloat16),
                   jax.ShapeDtypeStruct((N, HID), jnp.bfloat16)],
    )(h, wa, wb, b1.reshape(1, HID))


def _feat_kernel(h_ref, cw_ref, cwp_ref, dw_ref, db_ref,
                 f2_ref, if_ref, diss_ref):
    h = h_ref[...]
    if_ref[...] = _dott(h, cw_ref[...])
    fsc = _dott(h, cwp_ref[...]).astype(jnp.bfloat16)
    f2_ref[0] = fsc[:, :HH]
    f2_ref[1] = fsc[:, HH:]
    diss_ref[...] = jnp.maximum(_dott(h, dw_ref[...]) + db_ref[...], 0.0)


def _tc_feat(h, cw, cwp, dw, db):
    return pl.pallas_call(
        _feat_kernel,
        grid=(N // BN,),
        in_specs=[pl.BlockSpec((BN, HID), lambda i: (i, 0)),
                  pl.BlockSpec((HID, HID), lambda i: (0, 0)),
                  pl.BlockSpec((HID, HID), lambda i: (0, 0)),
                  pl.BlockSpec((HID, HID), lambda i: (0, 0)),
                  pl.BlockSpec((1, HID), lambda i: (0, 0))],
        out_specs=[pl.BlockSpec((2, BN, HH), lambda i: (0, i, 0)),
                   pl.BlockSpec((BN, HID), lambda i: (i, 0)),
                   pl.BlockSpec((BN, HID), lambda i: (i, 0))],
        out_shape=[jax.ShapeDtypeStruct((2, N, HH), jnp.bfloat16),
                   jax.ShapeDtypeStruct((N, HID), jnp.float32),
                   jax.ShapeDtypeStruct((N, HID), jnp.float32)],
    )(h, cw, cwp, dw, db.reshape(1, HID))


def _upd_kernel(h_ref, v_ref, if_ref, s2_ref, degp_ref, diss_ref,
                ho_ref, vo_ref):
    in_feat = if_ref[...]
    s = jnp.concatenate([s2_ref[0], s2_ref[1]], axis=1)
    deg = degp_ref[0, :, :1] + degp_ref[1, :, :1]
    conv = deg * in_feat - s
    v = v_ref[...]
    v = v - EPS * (conv + diss_ref[...] * v)
    ho_ref[...] = h_ref[...] + EPS * v
    vo_ref[...] = v


def _tc_update(h, v, in_feat, s2, degp, diss):
    return pl.pallas_call(
        _upd_kernel,
        grid=(N // BN,),
        in_specs=[pl.BlockSpec((BN, HID), lambda i: (i, 0)),
                  pl.BlockSpec((BN, HID), lambda i: (i, 0)),
                  pl.BlockSpec((BN, HID), lambda i: (i, 0)),
                  pl.BlockSpec((2, BN, HH), lambda i: (0, i, 0)),
                  pl.BlockSpec((2, BN, 16), lambda i: (0, i, 0)),
                  pl.BlockSpec((BN, HID), lambda i: (i, 0))],
        out_specs=[pl.BlockSpec((BN, HID), lambda i: (i, 0)),
                   pl.BlockSpec((BN, HID), lambda i: (i, 0))],
        out_shape=[jax.ShapeDtypeStruct((N, HID), jnp.float32),
                   jax.ShapeDtypeStruct((N, HID), jnp.float32)],
    )(h, v, in_feat, s2, degp, diss)


def _mlp_kernel(h_ref, w1_ref, b1_ref, w2_ref, b2_ref, o_ref):
    t = jnp.tanh(_dott(h_ref[...], w1_ref[...]) + b1_ref[...])
    o_ref[...] = _dott(t, w2_ref[...]) + b2_ref[...]


def _tc_mlp(h, w1, b1, w2, b2):
    return pl.pallas_call(
        _mlp_kernel,
        grid=(N // BN,),
        in_specs=[pl.BlockSpec((BN, HID), lambda i: (i, 0)),
                  pl.BlockSpec((HID, HID), lambda i: (0, 0)),
                  pl.BlockSpec((1, HID), lambda i: (0, 0)),
                  pl.BlockSpec((HID, HID), lambda i: (0, 0)),
                  pl.BlockSpec((1, HID), lambda i: (0, 0))],
        out_specs=pl.BlockSpec((BN, HID), lambda i: (i, 0)),
        out_shape=jax.ShapeDtypeStruct((N, HID), jnp.float32),
    )(h, w1, b1.reshape(1, HID), w2, b2.reshape(1, HID))


def _lrelu(x):
    return jnp.where(x >= 0.0, x, 0.01 * x)


def _ro_kernel(h_ref, w1_ref, b1_ref, w2_ref, b2_ref, o_ref):
    t = _lrelu(_dott(h_ref[...], w1_ref[...]) + b1_ref[...])
    o_ref[...] = _lrelu(_dott(t, w2_ref[...]) + b2_ref[...])


def _tc_readout(h, w1, b1, w2, b2):
    hh, od = w1.shape[0], w2.shape[0]
    return pl.pallas_call(
        _ro_kernel,
        grid=(N // BN,),
        in_specs=[pl.BlockSpec((BN, HID), lambda i: (i, 0)),
                  pl.BlockSpec((hh, HID), lambda i: (0, 0)),
                  pl.BlockSpec((1, hh), lambda i: (0, 0)),
                  pl.BlockSpec((od, hh), lambda i: (0, 0)),
                  pl.BlockSpec((1, od), lambda i: (0, 0))],
        out_specs=pl.BlockSpec((BN, od), lambda i: (i, 0)),
        out_shape=jax.ShapeDtypeStruct((N, od), jnp.float32),
    )(h, w1, b1.reshape(1, hh), w2, b2.reshape(1, od))


_GDN = lax.GatherDimensionNumbers(offset_dims=(), collapsed_slice_dims=(0,),
                                  start_index_map=(0,))


def _vperm(v, idx):
    return lax.gather(v, idx.reshape(16, 1), _GDN, (1,),
                      mode=lax.GatherScatterMode.PROMISE_IN_BOUNDS)


def _lane_bcast(v, k):
    return _vperm(v, jnp.full((16,), k, jnp.int32))


def _allsum(v):
    lane = lax.iota(jnp.int32, 16)
    for sh in (8, 4, 2, 1):
        v = v + _vperm(v, lane ^ sh)
    return v


_UNPACK = functools.partial(plsc.unpack, format=plsc.PackFormat.INTERLEAVED)


@functools.partial(
    pl.kernel, mesh=_mesh, compiler_params=_scp,
    out_type=[jax.ShapeDtypeStruct((E_PAD,), jnp.float32),
              jax.ShapeDtypeStruct((NC, N, 16), jnp.float32)],
    scratch_types=[
        pltpu.VMEM((CH1,), jnp.int32),
        pltpu.VMEM((CH1,), jnp.int32),
        pltpu.VMEM((CH1, HID), jnp.bfloat16),
        pltpu.VMEM((CH1, HID), jnp.bfloat16),
        pltpu.VMEM((CH1, HID), jnp.bfloat16),
        pltpu.VMEM((CH1,), jnp.int32),
        pltpu.VMEM((CH1,), jnp.int32),
        pltpu.VMEM((CH1, HID), jnp.bfloat16),
        pltpu.VMEM((CH1, HID), jnp.bfloat16),
        pltpu.VMEM((CH1, HID), jnp.bfloat16),
        pltpu.VMEM((CH1,), jnp.float32),
        pltpu.VMEM((CH1, 16), jnp.float32),
        pltpu.VMEM((HID,), jnp.float32),
        pltpu.VMEM((16,), jnp.float32),
        pltpu.VMEM_SHARED((N, 16), jnp.float32),
        pltpu.SemaphoreType.DMA,
        pltpu.SemaphoreType.DMA,
    ])
def _sc_res(p_hbm, q_hbm, r_hbm, row_hbm, col_hbm, w2_hbm, b2_hbm, zdeg_hbm,
            er_out, degp_out,
            idxr0, idxc0, bufP0, bufQ0, bufR0,
            idxr1, idxc1, bufP1, bufQ1, bufR1,
            bufE, bufD, w2v, b2v, degacc, gsem0, gsem1):
    c = lax.axis_index("c")
    s = lax.axis_index("s")
    wid = s * NC + c
    base = wid * EPW
    maxoff = base + (NCH1 - 1) * CH1

    pltpu.sync_copy(w2_hbm, w2v)
    pltpu.sync_copy(b2_hbm, b2v)

    @pl.when(s == 0)
    def _():
        pltpu.sync_copy(zdeg_hbm.at[c], degacc)

    def _zb(i, carry):
        bufD[i, :] = jnp.zeros((16,), jnp.float32)
        return carry
    lax.fori_loop(0, CH1, _zb, 0)

    w2r = [w2v[pl.ds(t * 16, 16)] for t in range(16)]
    b2r = b2v[...]

    bufs = [(idxr0, idxc0, bufP0, bufQ0, bufR0, gsem0),
            (idxr1, idxc1, bufP1, bufQ1, bufR1, gsem1)]

    def issue(j, p):
        idxr, idxc, bufP, bufQ, bufR, gsem = bufs[p]
        off = jnp.minimum(base + j * CH1, maxoff)
        pltpu.sync_copy(row_hbm.at[pl.ds(off, CH1)], idxr)
        pltpu.sync_copy(col_hbm.at[pl.ds(off, CH1)], idxc)
        pltpu.async_copy(p_hbm.at[idxr], bufP, gsem)
        pltpu.async_copy(q_hbm.at[idxc], bufQ, gsem)
        pltpu.async_copy(r_hbm.at[pl.ds(off, CH1), :], bufR, gsem)

    def drain(p):
        idxr, idxc, bufP, bufQ, bufR, gsem = bufs[p]
        pltpu.make_async_copy(p_hbm.at[pl.ds(0, CH1), :], bufP, gsem).wait()
        pltpu.make_async_copy(q_hbm.at[pl.ds(0, CH1), :], bufQ, gsem).wait()
        pltpu.make_async_copy(r_hbm.at[pl.ds(0, CH1), :], bufR, gsem).wait()

    def process(j, p):
        idxr, idxc, bufP, bufQ, bufR, gsem = bufs[p]
        off = base + j * CH1

        def _edge(k, carry2):
            acc = jnp.zeros((16,), jnp.float32)
            for g in range(8):
                sl = pl.ds(g * 32, 32)
                pa, pb = _UNPACK(bufP[k, sl])
                qa, qb = _UNPACK(bufQ[k, sl])
                ra, rb = _UNPACK(bufR[k, sl])
                za = jnp.maximum(pa + qa + ra, 0.0)
                zb = jnp.maximum(pb + qb + rb, 0.0)
                acc = acc + za * w2r[2 * g] + zb * w2r[2 * g + 1]
            erv = jnp.abs(_allsum(acc) + b2r)
            validf = jnp.where(off + k < E, 1.0, 0.0).astype(jnp.float32)
            erv = erv * validf
            idxk = jnp.full((16,), k, jnp.int32)
            m15 = lax.iota(jnp.int32, 16) == 15
            plsc.store_scatter(bufE, [idxk], erv, mask=m15)
            plsc.store_scatter(bufD, [idxk, jnp.zeros((16,), jnp.int32)],
                               erv, mask=m15)
            return carry2
        lax.fori_loop(0, CH1, _edge, 0)

        pltpu.sync_copy(bufE, er_out.at[pl.ds(off, CH1)])
        pltpu.sync_copy(bufD, degacc.at[idxr], add=True)

    plsc.subcore_barrier()

    issue(0, 0)

    def body(jj, carry):
        j0 = 2 * jj
        drain(0)
        issue(j0 + 1, 1)
        process(j0, 0)
        drain(1)
        issue(j0 + 2, 0)
        process(j0 + 1, 1)
        return carry
    lax.fori_loop(0, NCH1 // 2, body, 0)
    drain(0)

    plsc.subcore_barrier()

    @pl.when(s == 0)
    def _():
        pltpu.sync_copy(degacc, degp_out.at[c])


@functools.partial(
    pl.kernel, mesh=_mesh, compiler_params=_scp,
    out_type=jax.ShapeDtypeStruct((NC, N, HH), jnp.float32),
    scratch_types=[
        pltpu.VMEM((CH2,), jnp.int32),
        pltpu.VMEM((1, HH), jnp.int32),
        pltpu.VMEM((CH2, HH), jnp.bfloat16),
        pltpu.VMEM((CH2, HH), jnp.float32),
        pltpu.VMEM((CH2,), jnp.float32),
        pltpu.VMEM((CH2,), jnp.int32),
        pltpu.VMEM((1, HH), jnp.int32),
        pltpu.VMEM((CH2, HH), jnp.bfloat16),
        pltpu.VMEM((CH2, HH), jnp.float32),
        pltpu.VMEM((CH2,), jnp.float32),
        pltpu.VMEM((CH2,), jnp.int32),
        pltpu.VMEM_SHARED((N, HH), jnp.float32),
        pltpu.SemaphoreType.DMA,
        pltpu.SemaphoreType.DMA,
        pltpu.SemaphoreType.DMA,
        pltpu.SemaphoreType.DMA,
    ])
def _sc_scat(f2_hbm, er_hbm, row_hbm, col2_hbm, zs_hbm,
             s2_out,
             idxro0, idxc0, bufG0, bufS0, bufE0,
             idxro1, idxc1, bufG1, bufS1, bufE1,
             idxtmp, sacc, gsem0, gsem1, ssem0, ssem1):
    c = lax.axis_index("c")
    s = lax.axis_index("s")
    base = s * EPT
    maxoff = base + (NCH2 - 1) * CH2
    coff = c * N

    @pl.when(s == 0)
    def _():
        pltpu.sync_copy(zs_hbm.at[c], sacc)

    plsc.subcore_barrier()

    bufs = [(idxro0, idxc0, bufG0, bufS0, bufE0, gsem0, ssem0),
            (idxro1, idxc1, bufG1, bufS1, bufE1, gsem1, ssem1)]

    def gissue(j, p):
        idxro, idxc, bufG, bufS, bufE, gsem, ssem = bufs[p]
        off = jnp.minimum(base + j * CH2, maxoff)
        pltpu.sync_copy(row_hbm.at[pl.ds(off, CH2)], idxtmp)

        def _oi(t, carry2):
            sl = pl.ds(t * 16, 16)
            idxro[sl] = idxtmp[sl] + coff
            return carry2
        lax.fori_loop(0, CH2 // 16, _oi, 0)
        pltpu.async_copy(f2_hbm.at[idxro], bufG, gsem)

    def sload(j, p):
        idxro, idxc, bufG, bufS, bufE, gsem, ssem = bufs[p]
        off = base + j * CH2
        pltpu.sync_copy(col2_hbm.at[pl.ds(off // HH, 1), :], idxc)
        pltpu.sync_copy(er_hbm.at[pl.ds(off, CH2)], bufE)

    def gdrain(p):
        idxro, idxc, bufG, bufS, bufE, gsem, ssem = bufs[p]
        pltpu.make_async_copy(f2_hbm.at[pl.ds(0, CH2), :], bufG, gsem).wait()

    def sdrain(p):
        idxro, idxc, bufG, bufS, bufE, gsem, ssem = bufs[p]
        pltpu.make_async_copy(zs_hbm.at[0, pl.ds(0, CH2), :],
                              bufS, ssem).wait()

    def process(p):
        idxro, idxc, bufG, bufS, bufE, gsem, ssem = bufs[p]

        def _e16(t, carry2):
            ev = bufE[pl.ds(t * 16, 16)]
            for k in range(16):
                eb = _lane_bcast(ev, k)
                r = t * 16 + k
                for g in range(HH // 32):
                    a, b = _UNPACK(bufG[r, pl.ds(g * 32, 32)])
                    bufS[r, pl.ds(g * 32, 16)] = a * eb
                    bufS[r, pl.ds(g * 32 + 16, 16)] = b * eb
            return carry2
        lax.fori_loop(0, CH2 // 16, _e16, 0)
        pltpu.async_copy(bufS, sacc.at[idxc.at[0]], ssem, add=True)

    gissue(0, 0)
    gissue(1, 1)

    def body(jj, carry):
        j0 = 2 * jj
        gdrain(0)

        @pl.when(jj >= 1)
        def _():
            sdrain(0)
        sload(j0, 0)
        process(0)
        gissue(j0 + 2, 0)
        gdrain(1)

        @pl.when(jj >= 1)
        def _():
            sdrain(1)
        sload(j0 + 1, 1)
        process(1)
        gissue(j0 + 3, 1)
        return carry
    lax.fori_loop(0, NCH2 // 2, body, 0)
    gdrain(0)
    gdrain(1)
    sdrain(0)
    sdrain(1)

    plsc.subcore_barrier()

    @pl.when(s == 0)
    def _():
        pltpu.sync_copy(sacc, s2_out.at[c])


def kernel(x, edge_index, batch, edge_weight, emb_W, emb_b, conv_lin_W,
           ern_W1, ern_b1, ern_W2, ern_b2, diss_W, diss_b,
           mlp_W1, mlp_b1, mlp_W2, mlp_b2, ro_W1, ro_b1, ro_W2, ro_b2):
    pad = E_PAD - E
    rowp = jnp.pad(edge_index[0], (0, pad))
    colp = jnp.pad(edge_index[1], (0, pad))
    col2 = colp.reshape(E_PAD // HH, HH)
    ew8 = jnp.pad(edge_weight, ((0, pad), (0, 4)))
    zdeg = jnp.zeros((NC, N, 16), jnp.float32)
    zs = jnp.zeros((NC, N, HH), jnp.float32)
    perm = jnp.asarray(_PERM)
    permw = jnp.asarray(_PERM_W)

    h = _tc_linear(x, emb_W, emb_b, BN)
    for i in range(NUM_BLOCKS):
        w1 = ern_W1[i]
        wa = w1[:, :HID]
        wb = w1[:, HID:2 * HID]
        wc8 = jnp.pad(w1[:, 2 * HID:], ((0, 0), (0, 4)))
        p, q = _tc_pq(h, wa, wb, ern_b1[i])
        r = _tc_linear(ew8, wc8, jnp.zeros((HID,), jnp.float32), BM,
                       out_dtype=jnp.bfloat16)
        b2vec = jnp.broadcast_to(ern_b2[i, 0], (16,))
        w2p = ern_W2[i, 0][permw]
        er, degp = _sc_res(p, q, r, rowp, colp, w2p, b2vec, zdeg)

        cw = conv_lin_W[i]
        cwp = cw[perm]
        v = jnp.zeros((N, HID), jnp.float32)
        for _ in range(NUM_ITERS):
            f2, in_feat, diss = _tc_feat(h, cw, cwp, diss_W[i], diss_b[i])
            s2 = _sc_scat(f2.reshape(2 * N, HH), er, rowp, col2, zs)
            h, v = _tc_update(h, v, in_feat, s2, degp, diss)
        h = _tc_mlp(h, mlp_W1[i], mlp_b1[i], mlp_W2[i], mlp_b2[i])

    return _tc_readout(h, ro_W1, ro_b1, ro_W2, ro_b2)


# speedup vs baseline: 2.7126x; 1.1324x over previous
import functools

import jax
import jax.numpy as jnp
import numpy as np
from jax import lax
from jax.experimental import pallas as pl
from jax.experimental.pallas import tpu as pltpu
from jax.experimental.pallas import tpu_sc as plsc

N = 10000
E = 320000
HID = 256
OUT_DIM = 64
NUM_BLOCKS = 2
NUM_ITERS = 2
EPS = 0.1

NC = 2
NS = 16
NW = NC * NS

E_PAD = 327680
EPW = E_PAD // NW
CH1 = 128
NCH1 = EPW // CH1
EPT = E_PAD // NS
CH2 = 80
NCH2 = EPT // CH2
HH = HID // 2

BN = 2000
BM = 4096

_mesh = plsc.VectorSubcoreMesh(core_axis_name="c", subcore_axis_name="s")
_scp = pltpu.CompilerParams(needs_layout_passes=False,
                            use_tc_tiling_on_sc=False)

_k = np.arange(HID)
_m = _k % 32
_PERM = (_k - _m + 16 * (_m % 2) + _m // 2).astype(np.int32)
_PERM_W = (_k - _m + 2 * (_m % 16) + _m // 16).astype(np.int32)


def _dott(a, b):
    return lax.dot_general(a, b, (((1,), (1,)), ((), ())),
                           preferred_element_type=jnp.float32)


def _lin_kernel(x_ref, w_ref, b_ref, o_ref):
    y = _dott(x_ref[...], w_ref[...]) + b_ref[...]
    o_ref[...] = y.astype(o_ref.dtype)


def _tc_linear(x, w, b, bm, out_dtype=jnp.float32):
    m, k = x.shape
    o = w.shape[0]
    return pl.pallas_call(
        _lin_kernel,
        grid=(m // bm,),
        in_specs=[pl.BlockSpec((bm, k), lambda i: (i, 0)),
                  pl.BlockSpec((o, k), lambda i: (0, 0)),
                  pl.BlockSpec((1, o), lambda i: (0, 0))],
        out_specs=pl.BlockSpec((bm, o), lambda i: (i, 0)),
        out_shape=jax.ShapeDtypeStruct((m, o), out_dtype),
    )(x, w, b.reshape(1, o))


def _pq_kernel(h_ref, wa_ref, wb_ref, b1_ref, p_ref, q_ref):
    h = h_ref[...]
    p_ref[...] = (_dott(h, wa_ref[...]) + b1_ref[...]).astype(jnp.bfloat16)
    q_ref[...] = _dott(h, wb_ref[...]).astype(jnp.bfloat16)


def _tc_pq(h, wa, wb, b1):
    return pl.pallas_call(
        _pq_kernel,
        grid=(N // BN,),
        in_specs=[pl.BlockSpec((BN, HID), lambda i: (i, 0)),
                  pl.BlockSpec((HID, HID), lambda i: (0, 0)),
                  pl.BlockSpec((HID, HID), lambda i: (0, 0)),
                  pl.BlockSpec((1, HID), lambda i: (0, 0))],
        out_specs=[pl.BlockSpec((BN, HID), lambda i: (i, 0)),
                   pl.BlockSpec((BN, HID), lambda i: (i, 0))],
        out_shape=[jax.ShapeDtypeStruct((N, HID), jnp.bfloat16),
                   jax.ShapeDtypeStruct((N, HID), jnp.bfloat16)],
    )(h, wa, wb, b1.reshape(1, HID))


def _feat_kernel(h_ref, cw_ref, cwp_ref, dw_ref, db_ref,
                 f2_ref, if_ref, diss_ref):
    h = h_ref[...]
    if_ref[...] = _dott(h, cw_ref[...])
    fsc = _dott(h, cwp_ref[...]).astype(jnp.bfloat16)
    f2_ref[0] = fsc[:, :HH]
    f2_ref[1] = fsc[:, HH:]
    diss_ref[...] = jnp.maximum(_dott(h, dw_ref[...]) + db_ref[...], 0.0)


def _tc_feat(h, cw, cwp, dw, db):
    return pl.pallas_call(
        _feat_kernel,
        grid=(N // BN,),
        in_specs=[pl.BlockSpec((BN, HID), lambda i: (i, 0)),
                  pl.BlockSpec((HID, HID), lambda i: (0, 0)),
                  pl.BlockSpec((HID, HID), lambda i: (0, 0)),
                  pl.BlockSpec((HID, HID), lambda i: (0, 0)),
                  pl.BlockSpec((1, HID), lambda i: (0, 0))],
        out_specs=[pl.BlockSpec((2, BN, HH), lambda i: (0, i, 0)),
                   pl.BlockSpec((BN, HID), lambda i: (i, 0)),
                   pl.BlockSpec((BN, HID), lambda i: (i, 0))],
        out_shape=[jax.ShapeDtypeStruct((2, N, HH), jnp.bfloat16),
                   jax.ShapeDtypeStruct((N, HID), jnp.float32),
                   jax.ShapeDtypeStruct((N, HID), jnp.float32)],
    )(h, cw, cwp, dw, db.reshape(1, HID))


def _upd_kernel(h_ref, v_ref, if_ref, s2_ref, degp_ref, diss_ref,
                ho_ref, vo_ref):
    in_feat = if_ref[...]
    s = jnp.concatenate([s2_ref[0], s2_ref[1]], axis=1)
    deg = degp_ref[0, :, :1] + degp_ref[1, :, :1]
    conv = deg * in_feat - s
    v = v_ref[...]
    v = v - EPS * (conv + diss_ref[...] * v)
    ho_ref[...] = h_ref[...] + EPS * v
    vo_ref[...] = v


def _tc_update(h, v, in_feat, s2, degp, diss):
    return pl.pallas_call(
        _upd_kernel,
        grid=(N // BN,),
        in_specs=[pl.BlockSpec((BN, HID), lambda i: (i, 0)),
                  pl.BlockSpec((BN, HID), lambda i: (i, 0)),
                  pl.BlockSpec((BN, HID), lambda i: (i, 0)),
                  pl.BlockSpec((2, BN, HH), lambda i: (0, i, 0)),
                  pl.BlockSpec((2, BN, 16), lambda i: (0, i, 0)),
                  pl.BlockSpec((BN, HID), lambda i: (i, 0))],
        out_specs=[pl.BlockSpec((BN, HID), lambda i: (i, 0)),
                   pl.BlockSpec((BN, HID), lambda i: (i, 0))],
        out_shape=[jax.ShapeDtypeStruct((N, HID), jnp.float32),
                   jax.ShapeDtypeStruct((N, HID), jnp.float32)],
    )(h, v, in_feat, s2, degp, diss)


def _mlp_kernel(h_ref, w1_ref, b1_ref, w2_ref, b2_ref, o_ref):
    t = jnp.tanh(_dott(h_ref[...], w1_ref[...]) + b1_ref[...])
    o_ref[...] = _dott(t, w2_ref[...]) + b2_ref[...]


def _tc_mlp(h, w1, b1, w2, b2):
    return pl.pallas_call(
        _mlp_kernel,
        grid=(N // BN,),
        in_specs=[pl.BlockSpec((BN, HID), lambda i: (i, 0)),
                  pl.BlockSpec((HID, HID), lambda i: (0, 0)),
                  pl.BlockSpec((1, HID), lambda i: (0, 0)),
                  pl.BlockSpec((HID, HID), lambda i: (0, 0)),
                  pl.BlockSpec((1, HID), lambda i: (0, 0))],
        out_specs=pl.BlockSpec((BN, HID), lambda i: (i, 0)),
        out_shape=jax.ShapeDtypeStruct((N, HID), jnp.float32),
    )(h, w1, b1.reshape(1, HID), w2, b2.reshape(1, HID))


def _lrelu(x):
    return jnp.where(x >= 0.0, x, 0.01 * x)


def _ro_kernel(h_ref, w1_ref, b1_ref, w2_ref, b2_ref, o_ref):
    t = _lrelu(_dott(h_ref[...], w1_ref[...]) + b1_ref[...])
    o_ref[...] = _lrelu(_dott(t, w2_ref[...]) + b2_ref[...])


def _tc_readout(h, w1, b1, w2, b2):
    hh, od = w1.shape[0], w2.shape[0]
    return pl.pallas_call(
        _ro_kernel,
        grid=(N // BN,),
        in_specs=[pl.BlockSpec((BN, HID), lambda i: (i, 0)),
                  pl.BlockSpec((hh, HID), lambda i: (0, 0)),
                  pl.BlockSpec((1, hh), lambda i: (0, 0)),
                  pl.BlockSpec((od, hh), lambda i: (0, 0)),
                  pl.BlockSpec((1, od), lambda i: (0, 0))],
        out_specs=pl.BlockSpec((BN, od), lambda i: (i, 0)),
        out_shape=jax.ShapeDtypeStruct((N, od), jnp.float32),
    )(h, w1, b1.reshape(1, hh), w2, b2.reshape(1, od))


_GDN = lax.GatherDimensionNumbers(offset_dims=(), collapsed_slice_dims=(0,),
                                  start_index_map=(0,))


def _vperm(v, idx):
    return lax.gather(v, idx.reshape(16, 1), _GDN, (1,),
                      mode=lax.GatherScatterMode.PROMISE_IN_BOUNDS)


def _lane_bcast(v, k):
    return _vperm(v, jnp.full((16,), k, jnp.int32))


def _allsum(v):
    lane = lax.iota(jnp.int32, 16)
    for sh in (8, 4, 2, 1):
        v = v + _vperm(v, lane ^ sh)
    return v


_UNPACK = functools.partial(plsc.unpack, format=plsc.PackFormat.INTERLEAVED)


SB1 = 16
NSB1 = NCH1 // SB1


@functools.partial(
    pl.kernel, mesh=_mesh, compiler_params=_scp,
    out_type=[jax.ShapeDtypeStruct((E_PAD,), jnp.float32),
              jax.ShapeDtypeStruct((NC, N, 16), jnp.float32)],
    scratch_types=[
        pltpu.VMEM((CH1, HID), jnp.bfloat16),
        pltpu.VMEM((CH1, HID), jnp.bfloat16),
        pltpu.VMEM((CH1, HID), jnp.bfloat16),
        pltpu.VMEM((CH1, HID), jnp.bfloat16),
        pltpu.VMEM((CH1, HID), jnp.bfloat16),
        pltpu.VMEM((CH1, HID), jnp.bfloat16),
        pltpu.VMEM((SB1, CH1), jnp.int32),
        pltpu.VMEM((SB1, CH1), jnp.int32),
        pltpu.VMEM((SB1 * CH1,), jnp.float32),
        pltpu.VMEM((CH1, 16), jnp.float32),
        pltpu.VMEM((CH1, 16), jnp.float32),
        pltpu.VMEM((HID,), jnp.float32),
        pltpu.VMEM((16,), jnp.float32),
        pltpu.VMEM_SHARED((N, 16), jnp.float32),
        pltpu.SemaphoreType.DMA,
        pltpu.SemaphoreType.DMA,
        pltpu.SemaphoreType.DMA,
        pltpu.SemaphoreType.DMA,
    ])
def _sc_res(p_hbm, q_hbm, r_hbm, row2_hbm, col2_hbm, w2_hbm, b2_hbm,
            zdeg_hbm, er_out, degp_out,
            bufP0, bufQ0, bufR0, bufP1, bufQ1, bufR1,
            rowblk, colblk, bufE, bufD0, bufD1, w2v, b2v, degacc,
            gsem0, gsem1, dsem0, dsem1):
    c = lax.axis_index("c")
    s = lax.axis_index("s")
    wid = s * NC + c
    base = wid * EPW

    pltpu.sync_copy(w2_hbm, w2v)
    pltpu.sync_copy(b2_hbm, b2v)

    @pl.when(s == 0)
    def _():
        pltpu.sync_copy(zdeg_hbm.at[c], degacc)

    for bufD in (bufD0, bufD1):
        def _zb(i, carry, bufD=bufD):
            bufD[i, :] = jnp.zeros((16,), jnp.float32)
            return carry
        lax.fori_loop(0, CH1, _zb, 0)

    w2r = [w2v[pl.ds(t * 16, 16)] for t in range(16)]
    b2r = b2v[...]

    bufs = [(bufP0, bufQ0, bufR0, bufD0, gsem0, dsem0),
            (bufP1, bufQ1, bufR1, bufD1, gsem1, dsem1)]

    def gissue(sboff, cc, p):
        bufP, bufQ, bufR, bufD, gsem, dsem = bufs[p]
        pltpu.async_copy(p_hbm.at[rowblk.at[cc]], bufP, gsem)
        pltpu.async_copy(q_hbm.at[colblk.at[cc]], bufQ, gsem)
        pltpu.async_copy(r_hbm.at[pl.ds(sboff + cc * CH1, CH1), :],
                         bufR, gsem)

    def gdrain(p):
        bufP, bufQ, bufR, bufD, gsem, dsem = bufs[p]
        pltpu.make_async_copy(p_hbm.at[pl.ds(0, CH1), :], bufP, gsem).wait()
        pltpu.make_async_copy(q_hbm.at[pl.ds(0, CH1), :], bufQ, gsem).wait()
        pltpu.make_async_copy(r_hbm.at[pl.ds(0, CH1), :], bufR, gsem).wait()

    def ddrain(p):
        bufP, bufQ, bufR, bufD, gsem, dsem = bufs[p]
        pltpu.make_async_copy(zdeg_hbm.at[0, pl.ds(0, CH1), :],
                              bufD, dsem).wait()

    def process(sboff, cc, p):
        bufP, bufQ, bufR, bufD, gsem, dsem = bufs[p]
        off = sboff + cc * CH1

        def _edge(k, carry2):
            acc = jnp.zeros((16,), jnp.float32)
            for g in range(8):
                sl = pl.ds(g * 32, 32)
                pa, pb = _UNPACK(bufP[k, sl])
                qa, qb = _UNPACK(bufQ[k, sl])
                ra, rb = _UNPACK(bufR[k, sl])
                za = jnp.maximum(pa + qa + ra, 0.0)
                zb = jnp.maximum(pb + qb + rb, 0.0)
                acc = acc + za * w2r[2 * g] + zb * w2r[2 * g + 1]
            erv = jnp.abs(_allsum(acc) + b2r)
            validf = jnp.where(off + k < E, 1.0, 0.0).astype(jnp.float32)
            erv = erv * validf
            idxk = jnp.full((16,), cc * CH1 + k, jnp.int32)
            idxk2 = jnp.full((16,), k, jnp.int32)
            m15 = lax.iota(jnp.int32, 16) == 15
            plsc.store_scatter(bufE, [idxk], erv, mask=m15)
            plsc.store_scatter(bufD, [idxk2, jnp.zeros((16,), jnp.int32)],
                               erv, mask=m15)
            return carry2
        lax.fori_loop(0, CH1, _edge, 0)

        pltpu.async_copy(bufD, degacc.at[rowblk.at[cc]], dsem, add=True)

    plsc.subcore_barrier()

    def sblock(sb, carry):
        sboff = base + sb * SB1 * CH1

        @pl.when(sb >= 1)
        def _():
            ddrain(0)
            ddrain(1)
        srow = (base // CH1) + sb * SB1
        pltpu.sync_copy(row2_hbm.at[pl.ds(srow, SB1), :], rowblk)
        pltpu.sync_copy(col2_hbm.at[pl.ds(srow, SB1), :], colblk)

        gissue(sboff, 0, 0)
        gissue(sboff, 1, 1)

        def inner(ii, carry2):
            cc0 = 2 * ii
            for ph in range(2):
                cc = cc0 + ph
                gdrain(ph)

                @pl.when(ii >= 1)
                def _():
                    ddrain(ph)
                process(sboff, cc, ph)

                @pl.when(cc + 2 < SB1)
                def _():
                    gissue(sboff, cc + 2, ph)
            return carry2
        lax.fori_loop(0, SB1 // 2, inner, 0)

        pltpu.sync_copy(bufE, er_out.at[pl.ds(sboff, SB1 * CH1)])
        return carry
    lax.fori_loop(0, NSB1, sblock, 0)
    ddrain(0)
    ddrain(1)

    plsc.subcore_barrier()

    @pl.when(s == 0)
    def _():
        pltpu.sync_copy(degacc, degp_out.at[c])


SB2 = 16
NSB2 = NCH2 // SB2


@functools.partial(
    pl.kernel, mesh=_mesh, compiler_params=_scp,
    out_type=jax.ShapeDtypeStruct((NC, N, HH), jnp.float32),
    scratch_types=[
        pltpu.VMEM((CH2, HH), jnp.bfloat16),
        pltpu.VMEM((CH2, HH), jnp.float32),
        pltpu.VMEM((CH2, HH), jnp.bfloat16),
        pltpu.VMEM((CH2, HH), jnp.float32),
        pltpu.VMEM((SB2, CH2), jnp.int32),
        pltpu.VMEM((SB2, CH2), jnp.int32),
        pltpu.VMEM((SB2 * CH2,), jnp.float32),
        pltpu.VMEM_SHARED((N, HH), jnp.float32),
        pltpu.SemaphoreType.DMA,
        pltpu.SemaphoreType.DMA,
        pltpu.SemaphoreType.DMA,
        pltpu.SemaphoreType.DMA,
    ])
def _sc_scat(f2_hbm, er_hbm, row2_hbm, col2_hbm, zs_hbm,
             s2_out,
             bufG0, bufS0, bufG1, bufS1,
             rowoblk, colblk, erblk, sacc, gsem0, gsem1, ssem0, ssem1):
    c = lax.axis_index("c")
    s = lax.axis_index("s")
    base = s * EPT
    coff = c * N

    @pl.when(s == 0)
    def _():
        pltpu.sync_copy(zs_hbm.at[c], sacc)

    plsc.subcore_barrier()

    bufs = [(bufG0, bufS0, gsem0, ssem0),
            (bufG1, bufS1, gsem1, ssem1)]

    def gissue(cc, p):
        bufG, bufS, gsem, ssem = bufs[p]
        pltpu.async_copy(f2_hbm.at[rowoblk.at[cc]], bufG, gsem)

    def gdrain(p):
        bufG, bufS, gsem, ssem = bufs[p]
        pltpu.make_async_copy(f2_hbm.at[pl.ds(0, CH2), :], bufG, gsem).wait()

    def sdrain(p):
        bufG, bufS, gsem, ssem = bufs[p]
        pltpu.make_async_copy(zs_hbm.at[0, pl.ds(0, CH2), :],
                              bufS, ssem).wait()

    def process(cc, p):
        bufG, bufS, gsem, ssem = bufs[p]

        def _e16(t, carry2):
            ev = erblk[pl.ds(cc * CH2 + t * 16, 16)]
            for k in range(16):
                eb = _lane_bcast(ev, k)
                r = t * 16 + k
                for g in range(HH // 32):
                    a, b = _UNPACK(bufG[r, pl.ds(g * 32, 32)])
                    bufS[r, pl.ds(g * 32, 16)] = a * eb
                    bufS[r, pl.ds(g * 32 + 16, 16)] = b * eb
            return carry2
        lax.fori_loop(0, CH2 // 16, _e16, 0)
        pltpu.async_copy(bufS, sacc.at[colblk.at[cc]], ssem, add=True)

    def sblock(sb, carry):
        sboff = base + sb * SB2 * CH2

        @pl.when(sb >= 1)
        def _():
            sdrain(0)
            sdrain(1)
        srow = (base // CH2) + sb * SB2
        pltpu.sync_copy(row2_hbm.at[pl.ds(srow, SB2), :], rowoblk)
        pltpu.sync_copy(col2_hbm.at[pl.ds(srow, SB2), :], colblk)
        pltpu.sync_copy(er_hbm.at[pl.ds(sboff, SB2 * CH2)], erblk)

        def _oi(t, carry2):
            u = t // (CH2 // 16)
            w = t % (CH2 // 16)
            sl = pl.ds(w * 16, 16)
            rowoblk[u, sl] = rowoblk[u, sl] + coff
            return carry2
        lax.fori_loop(0, SB2 * (CH2 // 16), _oi, 0)

        gissue(0, 0)
        gissue(1, 1)

        def inner(ii, carry2):
            cc0 = 2 * ii
            for ph in range(2):
                cc = cc0 + ph
                gdrain(ph)

                @pl.when(ii >= 1)
                def _():
                    sdrain(ph)
                process(cc, ph)

                @pl.when(cc + 2 < SB2)
                def _():
                    gissue(cc + 2, ph)
            return carry2
        lax.fori_loop(0, SB2 // 2, inner, 0)
        return carry
    lax.fori_loop(0, NSB2, sblock, 0)
    sdrain(0)
    sdrain(1)

    plsc.subcore_barrier()

    @pl.when(s == 0)
    def _():
        pltpu.sync_copy(sacc, s2_out.at[c])


def kernel(x, edge_index, batch, edge_weight, emb_W, emb_b, conv_lin_W,
           ern_W1, ern_b1, ern_W2, ern_b2, diss_W, diss_b,
           mlp_W1, mlp_b1, mlp_W2, mlp_b2, ro_W1, ro_b1, ro_W2, ro_b2):
    pad = E_PAD - E
    rowp = jnp.pad(edge_index[0], (0, pad))
    colp = jnp.pad(edge_index[1], (0, pad))
    row2 = rowp.reshape(E_PAD // CH1, CH1)
    col2 = colp.reshape(E_PAD // CH1, CH1)
    row2b = rowp.reshape(E_PAD // CH2, CH2)
    col2b = colp.reshape(E_PAD // CH2, CH2)
    ew8 = jnp.pad(edge_weight, ((0, pad), (0, 4)))
    zdeg = jnp.zeros((NC, N, 16), jnp.float32)
    zs = jnp.zeros((NC, N, HH), jnp.float32)
    perm = jnp.asarray(_PERM)
    permw = jnp.asarray(_PERM_W)

    h = _tc_linear(x, emb_W, emb_b, BN)
    for i in range(NUM_BLOCKS):
        w1 = ern_W1[i]
        wa = w1[:, :HID]
        wb = w1[:, HID:2 * HID]
        wc8 = jnp.pad(w1[:, 2 * HID:], ((0, 0), (0, 4)))
        p, q = _tc_pq(h, wa, wb, ern_b1[i])
        r = _tc_linear(ew8, wc8, jnp.zeros((HID,), jnp.float32), BM,
                       out_dtype=jnp.bfloat16)
        b2vec = jnp.broadcast_to(ern_b2[i, 0], (16,))
        w2p = ern_W2[i, 0][permw]
        er, degp = _sc_res(p, q, r, row2, col2, w2p, b2vec, zdeg)

        cw = conv_lin_W[i]
        cwp = cw[perm]
        v = jnp.zeros((N, HID), jnp.float32)
        for _ in range(NUM_ITERS):
            f2, in_feat, diss = _tc_feat(h, cw, cwp, diss_W[i], diss_b[i])
            s2 = _sc_scat(f2.reshape(2 * N, HH), er, row2b, col2b, zs)
            h, v = _tc_update(h, v, in_feat, s2, degp, diss)
        h = _tc_mlp(h, mlp_W1[i], mlp_b1[i], mlp_W2[i], mlp_b2[i])

    return _tc_readout(h, ro_W1, ro_b1, ro_W2, ro_b2)
